# baseline (speedup 1.0000x reference)
import functools

import jax
import jax.numpy as jnp
from jax import lax
from jax.experimental import pallas as pl
from jax.experimental.pallas import tpu as pltpu
from jax.experimental.pallas import tpu_sc as plsc

_NE = 1600000
_NW = 32
_SB = 512
_NSB = _NE // _SB
_NBQ = _NSB // _NW
_REM = _NSB - _NBQ * _NW
_CH = 128
_W = 4
_L = 16

_PI = 3.14159265358979
_HALF_PI = 1.57079632679490

_ATAN_C = (0.9999961118213437, -0.3331736830886415, 0.1980781555459296,
           -0.13233337654657124, 0.07962354669278539, -0.03360408888071814,
           0.006811745203309821)


def _rsqrt(s):
    i = lax.bitcast_convert_type(s, jnp.int32)
    i = jnp.int32(0x5F3759DF) - lax.shift_right_logical(i, 1)
    y = lax.bitcast_convert_type(i, jnp.float32)
    for _ in range(3):
        y = y * (1.5 - 0.5 * s * y * y)
    return y


def _sqrt(s):
    return s * _rsqrt(s)


def _atan2_pos(y, x):
    ax = jnp.abs(x)
    num = jnp.minimum(ax, y)
    den = jnp.maximum(jnp.maximum(ax, y), 1e-30)
    a = num / den
    z = a * a
    p = jnp.float32(_ATAN_C[-1])
    for c in _ATAN_C[-2::-1]:
        p = p * z + c
    t = a * p
    t = jnp.where(y > ax, _HALF_PI - t, t)
    t = jnp.where(x < 0.0, _PI - t, t)
    return t


def _sub(p, q):
    return (p[0] - q[0], p[1] - q[1], p[2] - q[2])


def _dot(u, v):
    return u[0] * v[0] + u[1] * v[1] + u[2] * v[2]


def _cross(u, v):
    return (u[1] * v[2] - u[2] * v[1],
            u[2] * v[0] - u[0] * v[2],
            u[0] * v[1] - u[1] * v[0])


def _dist(p, q):
    d = _sub(p, q)
    return _sqrt(_dot(d, d))


def _bond_math(pts):
    return (_dist(pts[0], pts[1]),)


def _angle_math(pts):
    p0, p1, p2 = pts
    r0 = _sub(p0, p1)
    r1 = _sub(p2, p1)
    cr = _cross(r0, r1)
    ang = _atan2_pos(_sqrt(_dot(cr, cr)), _dot(r0, r1))
    left = _sqrt(_dot(r0, r0))
    right = _sqrt(_dot(r1, r1))
    between = _dist(p0, p2)
    return (ang, left, right, between)


def _torsion_math(pts):
    p0, p1, p2, p3 = pts
    a = _sub(p1, p0)
    b = _sub(p1, p2)
    c = _sub(p2, p1)
    d = _sub(p2, p3)
    left = _cross(a, b)
    right = _cross(c, d)
    lr = _cross(left, right)
    tor = _atan2_pos(_sqrt(_dot(lr, lr)), _dot(left, right))
    bl = _sqrt(_dot(a, a))
    bc = _sqrt(_dot(c, c))
    brv = _sub(p3, p2)
    br = _sqrt(_dot(brv, brv))
    al = _atan2_pos(_sqrt(_dot(left, left)), _dot(a, b))
    ar = _atan2_pos(_sqrt(_dot(right, right)), _dot(c, d))
    return (tor, bl, bc, br, al, ar)


def _geom_body(px, py, pz, bond, angle, torsion, nonbonded, onefour,
               out, raw2, raw3, raw4, colbuf, rows, outb, sem):
    wid = lax.axis_index("s") * 2 + lax.axis_index("c")
    planes = (px, py, pz)
    nb = jnp.where(wid < _REM, _NBQ + 1, _NBQ)
    start_sb = wid * _NBQ + jnp.minimum(wid, _REM)

    lanes = lax.iota(jnp.int32, _L)
    cols = tuple(jnp.full((_L,), c, jnp.int32) for c in range(4))

    def process(idx_hbm, rawk, k, sections, mathfn):
        nch = k * (_SB // _CH)

        def gather_chunk(cc, start):
            for p in range(3):
                src = planes[p].at[colbuf.at[pl.ds(cc * _CH, _CH)]]
                dst = rows.at[p, pl.ds(cc * _CH, _CH)]
                if start:
                    pltpu.async_copy(src, dst, sem)
                else:
                    pltpu.make_async_copy(src, dst, sem).wait()

        def blk_body(s, carry):
            ebase = (start_sb + s) * _SB
            pltpu.sync_copy(idx_hbm.at[pl.ds(ebase, _SB), :], rawk)

            def extract(g, c2):
                rid = g * _L + lanes
                for j in range(k):
                    v = plsc.load_gather(rawk, [rid, cols[j]])
                    colbuf[pl.ds(j * _SB + g * _L, _L)] = v
                return c2
            lax.fori_loop(0, _SB // _L, extract, 0)

            def fire(cc, c2):
                gather_chunk(cc, True)
                @pl.when(cc >= _W)
                def _():
                    gather_chunk(cc - _W, False)
                return c2
            lax.fori_loop(0, nch, fire, 0)

            def drain(cc, c2):
                gather_chunk(cc, False)
                return c2
            lax.fori_loop(nch - _W, nch, drain, 0)

            def grp(g, c2):
                pts = []
                for j in range(k):
                    off = j * _SB + g * _L
                    pts.append(tuple(rows[p, pl.ds(off, _L)]
                                     for p in range(3)))
                vals = mathfn(pts)
                for o, v in enumerate(vals):
                    outb[o, pl.ds(g * _L, _L)] = v
                return c2
            lax.fori_loop(0, _SB // _L, grp, 0)

            for o, sect in enumerate(sections):
                pltpu.sync_copy(outb.at[o, pl.ds(0, _SB)],
                                out.at[pl.ds(sect * _NE + ebase, _SB)])
            return carry

        lax.fori_loop(0, nb, blk_body, 0)

    process(bond, raw2, 2, (0,), _bond_math)
    process(angle, raw3, 3, (1, 2, 3, 4), _angle_math)
    process(torsion, raw4, 4, (5, 6, 7, 8, 9, 10), _torsion_math)
    process(nonbonded, raw2, 2, (11,), _bond_math)
    process(onefour, raw2, 2, (12,), _bond_math)


@functools.cache
def _build_geom():
    return functools.partial(
        pl.kernel,
        out_type=jax.ShapeDtypeStruct((13 * _NE,), jnp.float32),
        mesh=plsc.VectorSubcoreMesh(core_axis_name="c", subcore_axis_name="s"),
        compiler_params=pltpu.CompilerParams(needs_layout_passes=False,
                                             use_tc_tiling_on_sc=False),
        scratch_types=[
            pltpu.VMEM((_SB, 2), jnp.int32),
            pltpu.VMEM((_SB, 3), jnp.int32),
            pltpu.VMEM((_SB, 4), jnp.int32),
            pltpu.VMEM((4 * _SB,), jnp.int32),
            pltpu.VMEM((3, 4 * _SB), jnp.float32),
            pltpu.VMEM((6, _SB), jnp.float32),
            pltpu.SemaphoreType.DMA,
        ],
    )(_geom_body)


def kernel(xyz, bond_idx, angle_idx, torsion_idx, nonbonded_idx, onefour_idx):
    return _build_geom()(xyz[:, 0], xyz[:, 1], xyz[:, 2],
                         bond_idx, angle_idx, torsion_idx,
                         nonbonded_idx, onefour_idx)


# speedup vs baseline: 3.7462x; 3.7462x over previous
import functools

import jax
import jax.numpy as jnp
from jax import lax
from jax.experimental import pallas as pl
from jax.experimental.pallas import tpu as pltpu
from jax.experimental.pallas import tpu_sc as plsc

_NE = 1600000
_NW = 32
_SB = 512
_NSB = _NE // _SB
_NBQ = _NSB // _NW
_REM = _NSB - _NBQ * _NW
_CH = 128
_W = 4
_L = 16

_PI = 3.14159265358979
_HALF_PI = 1.57079632679490

_ATAN_C = (0.9999961118213437, -0.3331736830886415, 0.1980781555459296,
           -0.13233337654657124, 0.07962354669278539, -0.03360408888071814,
           0.006811745203309821)


def _rsqrt(s):
    i = lax.bitcast_convert_type(s, jnp.int32)
    i = jnp.int32(0x5F3759DF) - lax.shift_right_logical(i, 1)
    y = lax.bitcast_convert_type(i, jnp.float32)
    for _ in range(3):
        y = y * (1.5 - 0.5 * s * y * y)
    return y


def _sqrt(s):
    return s * _rsqrt(s)


def _atan2_pos(y, x):
    ax = jnp.abs(x)
    num = jnp.minimum(ax, y)
    den = jnp.maximum(jnp.maximum(ax, y), 1e-30)
    a = num / den
    z = a * a
    p = jnp.float32(_ATAN_C[-1])
    for c in _ATAN_C[-2::-1]:
        p = p * z + c
    t = a * p
    t = jnp.where(y > ax, _HALF_PI - t, t)
    t = jnp.where(x < 0.0, _PI - t, t)
    return t


def _sub(p, q):
    return (p[0] - q[0], p[1] - q[1], p[2] - q[2])


def _dot(u, v):
    return u[0] * v[0] + u[1] * v[1] + u[2] * v[2]


def _cross(u, v):
    return (u[1] * v[2] - u[2] * v[1],
            u[2] * v[0] - u[0] * v[2],
            u[0] * v[1] - u[1] * v[0])


def _dist(p, q):
    d = _sub(p, q)
    return _sqrt(_dot(d, d))


def _bond_math(pts):
    return (_dist(pts[0], pts[1]),)


def _angle_math(pts):
    p0, p1, p2 = pts
    r0 = _sub(p0, p1)
    r1 = _sub(p2, p1)
    cr = _cross(r0, r1)
    ang = _atan2_pos(_sqrt(_dot(cr, cr)), _dot(r0, r1))
    left = _sqrt(_dot(r0, r0))
    right = _sqrt(_dot(r1, r1))
    between = _dist(p0, p2)
    return (ang, left, right, between)


def _torsion_math(pts):
    p0, p1, p2, p3 = pts
    a = _sub(p1, p0)
    b = _sub(p1, p2)
    c = _sub(p2, p1)
    d = _sub(p2, p3)
    left = _cross(a, b)
    right = _cross(c, d)
    lr = _cross(left, right)
    tor = _atan2_pos(_sqrt(_dot(lr, lr)), _dot(left, right))
    bl = _sqrt(_dot(a, a))
    bc = _sqrt(_dot(c, c))
    brv = _sub(p3, p2)
    br = _sqrt(_dot(brv, brv))
    al = _atan2_pos(_sqrt(_dot(left, left)), _dot(a, b))
    ar = _atan2_pos(_sqrt(_dot(right, right)), _dot(c, d))
    return (tor, bl, bc, br, al, ar)


def _geom_body(px, py, pz, bond, angle, torsion, nonbonded, onefour,
               out, colbuf, rows, outb, sem):
    wid = lax.axis_index("s") * 2 + lax.axis_index("c")
    planes = (px, py, pz)
    nb = jnp.where(wid < _REM, _NBQ + 1, _NBQ)
    start_sb = wid * _NBQ + jnp.minimum(wid, _REM)

    def process(idx_hbm, k, sections, mathfn):
        nch = k * (_SB // _CH)

        def gather_chunk(cc, start):
            for p in range(3):
                src = planes[p].at[colbuf.at[pl.ds(cc * _CH, _CH)]]
                dst = rows.at[p, pl.ds(cc * _CH, _CH)]
                if start:
                    pltpu.async_copy(src, dst, sem)
                else:
                    pltpu.make_async_copy(src, dst, sem).wait()

        def blk_body(s, carry):
            ebase = (start_sb + s) * _SB
            for j in range(k):
                pltpu.sync_copy(idx_hbm.at[pl.ds(j * _NE + ebase, _SB)],
                                colbuf.at[pl.ds(j * _SB, _SB)])

            def fire(cc, c2):
                gather_chunk(cc, True)
                @pl.when(cc >= _W)
                def _():
                    gather_chunk(cc - _W, False)
                return c2
            lax.fori_loop(0, nch, fire, 0)

            def drain(cc, c2):
                gather_chunk(cc, False)
                return c2
            lax.fori_loop(nch - _W, nch, drain, 0)

            def grp(g, c2):
                pts = []
                for j in range(k):
                    off = j * _SB + g * _L
                    pts.append(tuple(rows[p, pl.ds(off, _L)]
                                     for p in range(3)))
                vals = mathfn(pts)
                for o, v in enumerate(vals):
                    outb[o, pl.ds(g * _L, _L)] = v
                return c2
            lax.fori_loop(0, _SB // _L, grp, 0)

            for o, sect in enumerate(sections):
                pltpu.sync_copy(outb.at[o, pl.ds(0, _SB)],
                                out.at[pl.ds(sect * _NE + ebase, _SB)])
            return carry

        lax.fori_loop(0, nb, blk_body, 0)

    process(bond, 2, (0,), _bond_math)
    process(angle, 3, (1, 2, 3, 4), _angle_math)
    process(torsion, 4, (5, 6, 7, 8, 9, 10), _torsion_math)
    process(nonbonded, 2, (11,), _bond_math)
    process(onefour, 2, (12,), _bond_math)


@functools.cache
def _build_geom():
    return functools.partial(
        pl.kernel,
        out_type=jax.ShapeDtypeStruct((13 * _NE,), jnp.float32),
        mesh=plsc.VectorSubcoreMesh(core_axis_name="c", subcore_axis_name="s"),
        compiler_params=pltpu.CompilerParams(needs_layout_passes=False,
                                             use_tc_tiling_on_sc=False),
        scratch_types=[
            pltpu.VMEM((4 * _SB,), jnp.int32),
            pltpu.VMEM((3, 4 * _SB), jnp.float32),
            pltpu.VMEM((6, _SB), jnp.float32),
            pltpu.SemaphoreType.DMA,
        ],
    )(_geom_body)


def _slotmajor(idx):
    return jnp.concatenate([idx[:, j] for j in range(idx.shape[1])])


def kernel(xyz, bond_idx, angle_idx, torsion_idx, nonbonded_idx, onefour_idx):
    return _build_geom()(xyz[:, 0], xyz[:, 1], xyz[:, 2],
                         _slotmajor(bond_idx), _slotmajor(angle_idx),
                         _slotmajor(torsion_idx), _slotmajor(nonbonded_idx),
                         _slotmajor(onefour_idx))


# speedup vs baseline: 5.1502x; 1.3748x over previous
import functools

import jax
import jax.numpy as jnp
from jax import lax
from jax.experimental import pallas as pl
from jax.experimental.pallas import tpu as pltpu
from jax.experimental.pallas import tpu_sc as plsc

_NE = 1600000
_NW = 32
_SB = 256
_NSB = _NE // _SB
_NBQ = _NSB // _NW
_REM = _NSB - _NBQ * _NW
_CH = 128
_L = 16
_CBS = 4 * _SB
_PS = 4 * _SB
_RPS = 3 * _PS
_OPS = 6 * _SB

_PI = 3.14159265358979
_HALF_PI = 1.57079632679490

_ATAN_C = (0.9999961118213437, -0.3331736830886415, 0.1980781555459296,
           -0.13233337654657124, 0.07962354669278539, -0.03360408888071814,
           0.006811745203309821)


def _rsqrt(s):
    i = lax.bitcast_convert_type(s, jnp.int32)
    i = jnp.int32(0x5F3759DF) - lax.shift_right_logical(i, 1)
    y = lax.bitcast_convert_type(i, jnp.float32)
    for _ in range(3):
        y = y * (1.5 - 0.5 * s * y * y)
    return y


def _sqrt(s):
    return s * _rsqrt(s)


def _atan2_pos(y, x):
    ax = jnp.abs(x)
    num = jnp.minimum(ax, y)
    den = jnp.maximum(jnp.maximum(ax, y), 1e-30)
    a = num / den
    z = a * a
    p = jnp.float32(_ATAN_C[-1])
    for c in _ATAN_C[-2::-1]:
        p = p * z + c
    t = a * p
    t = jnp.where(y > ax, _HALF_PI - t, t)
    t = jnp.where(x < 0.0, _PI - t, t)
    return t


def _sub(p, q):
    return (p[0] - q[0], p[1] - q[1], p[2] - q[2])


def _dot(u, v):
    return u[0] * v[0] + u[1] * v[1] + u[2] * v[2]


def _cross(u, v):
    return (u[1] * v[2] - u[2] * v[1],
            u[2] * v[0] - u[0] * v[2],
            u[0] * v[1] - u[1] * v[0])


def _dist(p, q):
    d = _sub(p, q)
    return _sqrt(_dot(d, d))


def _bond_math(pts):
    return (_dist(pts[0], pts[1]),)


def _angle_math(pts):
    p0, p1, p2 = pts
    r0 = _sub(p0, p1)
    r1 = _sub(p2, p1)
    cr = _cross(r0, r1)
    ang = _atan2_pos(_sqrt(_dot(cr, cr)), _dot(r0, r1))
    left = _sqrt(_dot(r0, r0))
    right = _sqrt(_dot(r1, r1))
    between = _dist(p0, p2)
    return (ang, left, right, between)


def _torsion_math(pts):
    p0, p1, p2, p3 = pts
    a = _sub(p1, p0)
    b = _sub(p1, p2)
    c = _sub(p2, p1)
    d = _sub(p2, p3)
    left = _cross(a, b)
    right = _cross(c, d)
    lr = _cross(left, right)
    tor = _atan2_pos(_sqrt(_dot(lr, lr)), _dot(left, right))
    bl = _sqrt(_dot(a, a))
    bc = _sqrt(_dot(c, c))
    brv = _sub(p3, p2)
    br = _sqrt(_dot(brv, brv))
    al = _atan2_pos(_sqrt(_dot(left, left)), _dot(a, b))
    ar = _atan2_pos(_sqrt(_dot(right, right)), _dot(c, d))
    return (tor, bl, bc, br, al, ar)


def _geom_body(px, py, pz, bond, angle, torsion, nonbonded, onefour,
               out, colbuf, rows, outb, sem_i, sem_g, sem_o):
    wid = lax.axis_index("s") * 2 + lax.axis_index("c")
    planes = (px, py, pz)
    nb = jnp.where(wid < _REM, _NBQ + 1, _NBQ)
    start_sb = wid * _NBQ + jnp.minimum(wid, _REM)

    def process(idx_hbm, k, sections, mathfn):
        nch = k * (_SB // _CH)

        def idx_dma(b, issue):
            ebase = (start_sb + b) * _SB
            cb = (b & 1) * _CBS
            for j in range(k):
                src = idx_hbm.at[pl.ds(j * _NE + ebase, _SB)]
                dst = colbuf.at[pl.ds(cb + j * _SB, _SB)]
                if issue:
                    pltpu.async_copy(src, dst, sem_i)
                else:
                    pltpu.make_async_copy(src, dst, sem_i).wait()

        def gather_dma(b, issue):
            cb = (b & 1) * _CBS
            rb = (b & 1) * _RPS
            for cc in range(nch):
                for p in range(3):
                    src = planes[p].at[colbuf.at[pl.ds(cb + cc * _CH, _CH)]]
                    dst = rows.at[pl.ds(rb + p * _PS + cc * _CH, _CH)]
                    if issue:
                        pltpu.async_copy(src, dst, sem_g)
                    else:
                        pltpu.make_async_copy(src, dst, sem_g).wait()

        def out_dma(b, issue):
            ebase = (start_sb + b) * _SB
            ob = (b & 1) * _OPS
            for o, sect in enumerate(sections):
                src = outb.at[pl.ds(ob + o * _SB, _SB)]
                dst = out.at[pl.ds(sect * _NE + ebase, _SB)]
                if issue:
                    pltpu.async_copy(src, dst, sem_o)
                else:
                    pltpu.make_async_copy(src, dst, sem_o).wait()

        def math_blk(b):
            rb = (b & 1) * _RPS
            ob = (b & 1) * _OPS

            def grp(g, c2):
                pts = []
                for j in range(k):
                    off = j * _SB + g * _L
                    pts.append(tuple(rows[pl.ds(rb + p * _PS + off, _L)]
                                     for p in range(3)))
                vals = mathfn(pts)
                for o, v in enumerate(vals):
                    outb[pl.ds(ob + o * _SB + g * _L, _L)] = v
                return c2
            lax.fori_loop(0, _SB // _L, grp, 0)

        idx_dma(0, True)

        def step(s, carry):
            @pl.when((s >= 1) & (s <= nb))
            def _():
                gather_dma(s - 1, False)
            @pl.when(s < nb)
            def _():
                idx_dma(s, False)
                gather_dma(s, True)
            @pl.when(s + 1 < nb)
            def _():
                idx_dma(s + 1, True)
            @pl.when((s >= 2) & (s <= nb + 1))
            def _():
                out_dma(s - 2, False)
            @pl.when((s >= 1) & (s <= nb))
            def _():
                math_blk(s - 1)
                out_dma(s - 1, True)
            return carry

        lax.fori_loop(0, nb + 2, step, 0)

    process(bond, 2, (0,), _bond_math)
    process(angle, 3, (1, 2, 3, 4), _angle_math)
    process(torsion, 4, (5, 6, 7, 8, 9, 10), _torsion_math)
    process(nonbonded, 2, (11,), _bond_math)
    process(onefour, 2, (12,), _bond_math)


@functools.cache
def _build_geom():
    return functools.partial(
        pl.kernel,
        out_type=jax.ShapeDtypeStruct((13 * _NE,), jnp.float32),
        mesh=plsc.VectorSubcoreMesh(core_axis_name="c", subcore_axis_name="s"),
        compiler_params=pltpu.CompilerParams(needs_layout_passes=False,
                                             use_tc_tiling_on_sc=False),
        scratch_types=[
            pltpu.VMEM((2 * _CBS,), jnp.int32),
            pltpu.VMEM((2 * _RPS,), jnp.float32),
            pltpu.VMEM((2 * _OPS,), jnp.float32),
            pltpu.SemaphoreType.DMA,
            pltpu.SemaphoreType.DMA,
            pltpu.SemaphoreType.DMA,
        ],
    )(_geom_body)


def _slotmajor(idx):
    return jnp.concatenate([idx[:, j] for j in range(idx.shape[1])])


def kernel(xyz, bond_idx, angle_idx, torsion_idx, nonbonded_idx, onefour_idx):
    return _build_geom()(xyz[:, 0], xyz[:, 1], xyz[:, 2],
                         _slotmajor(bond_idx), _slotmajor(angle_idx),
                         _slotmajor(torsion_idx), _slotmajor(nonbonded_idx),
                         _slotmajor(onefour_idx))
